# baseline (speedup 1.0000x reference)
import functools

import jax
import jax.numpy as jnp
from jax import lax
from jax.experimental import pallas as pl
from jax.experimental.pallas import tpu as pltpu
from jax.experimental.pallas import tpu_sc as plsc

_LB = 128


def _make_gather(B, H, V, D, NC, NS):
    NW = NC * NS
    blk_per_w = B // _LB // NW
    b_per_w = blk_per_w * _LB
    n_tiles = H * blk_per_w
    n_pairs = n_tiles // 2
    E8 = D // 8

    mesh = plsc.VectorSubcoreMesh(core_axis_name="c", subcore_axis_name="s")

    @functools.partial(
        pl.kernel,
        mesh=mesh,
        compiler_params=pltpu.CompilerParams(
            use_tc_tiling_on_sc=False, needs_layout_passes=False
        ),
        out_type=jax.ShapeDtypeStruct((H, E8, B // _LB, 8, _LB), jnp.float32),
        scratch_types=[
            pltpu.VMEM((H, b_per_w), jnp.int32),
            pltpu.VMEM((2, _LB, D), jnp.float32),
            pltpu.VMEM((2, E8, 8, _LB), jnp.float32),
            pltpu.SemaphoreType.DMA,
            pltpu.SemaphoreType.DMA,
            pltpu.SemaphoreType.DMA,
            pltpu.SemaphoreType.DMA,
        ],
    )
    def gather(xt_hbm, table_hbm, out_hbm, idx_v, rows_v, t_v, g0, g1, s0, s1):
        gsem = (g0, g1)
        ssem = (s0, s1)
        wid = lax.axis_index("s") * NC + lax.axis_index("c")
        bbase = wid * blk_per_w
        pltpu.sync_copy(
            xt_hbm.at[:, pl.ds(pl.multiple_of(wid * b_per_w, b_per_w), b_per_w)],
            idx_v,
        )

        lane = lax.iota(jnp.int32, 16)
        row_idx = [lane + (g * 16) for g in range(_LB // 16)]
        col_idx = [jnp.full((16,), e, jnp.int32) for e in range(D)]

        def tile_coords(t):
            h = lax.shift_right_logical(t, 2)
            blk = lax.bitwise_and(t, blk_per_w - 1)
            return h, blk

        def fire_gather(t, b):
            h, blk = tile_coords(t)
            pltpu.async_copy(
                table_hbm.at[idx_v.at[h, pl.ds(pl.multiple_of(blk * _LB, _LB), _LB)]],
                rows_v.at[b],
                gsem[b],
            )

        def wait_gather(b):
            pltpu.make_async_copy(
                table_hbm.at[pl.ds(0, _LB)], rows_v.at[b], gsem[b]
            ).wait()

        def transpose(b):
            for e in range(D):
                for g in range(_LB // 16):
                    v = plsc.load_gather(rows_v.at[b], [row_idx[g], col_idx[e]])
                    t_v[b, e // 8, e % 8, pl.ds(g * 16, 16)] = v

        def fire_store(t, b):
            h, blk = tile_coords(t)
            pltpu.async_copy(t_v.at[b], out_hbm.at[h, :, bbase + blk], ssem[b])

        def wait_store(b):
            pltpu.make_async_copy(
                t_v.at[b], out_hbm.at[0, :, 0], ssem[b]
            ).wait()

        fire_gather(0, 0)

        def pair(i, carry):
            t = pl.multiple_of(i * 2, 2)
            wait_gather(0)
            fire_gather(t + 1, 1)

            @pl.when(i > 0)
            def _():
                wait_store(0)

            transpose(0)
            fire_store(t, 0)

            wait_gather(1)

            @pl.when(i < n_pairs - 1)
            def _():
                fire_gather(t + 2, 0)

            @pl.when(i > 0)
            def _():
                wait_store(1)

            transpose(1)
            fire_store(t + 1, 1)
            return carry

        lax.fori_loop(0, n_pairs, pair, 0)
        wait_store(0)
        wait_store(1)

    return gather


def kernel(x, table):
    B, H = x.shape
    V, D = table.shape
    xt = x.T.astype(jnp.int32)
    info = plsc.get_sparse_core_info()
    gather = _make_gather(B, H, V, D, info.num_cores, info.num_subcores)
    out5 = gather(xt, table)
    return out5.transpose(2, 4, 0, 1, 3).reshape(B, H, D)


# speedup vs baseline: 1.4056x; 1.4056x over previous
import functools

import jax
import jax.numpy as jnp
from jax import lax
from jax.experimental import pallas as pl
from jax.experimental.pallas import tpu as pltpu
from jax.experimental.pallas import tpu_sc as plsc

_LB = 128


def _make_gather(B, H, V, D, NC, NS):
    NW = NC * NS
    blk_per_w = B // _LB // NW
    b_per_w = blk_per_w * _LB
    n_tiles = H * blk_per_w
    n_pairs = n_tiles // 2
    E8 = D // 8

    mesh = plsc.VectorSubcoreMesh(core_axis_name="c", subcore_axis_name="s")

    @functools.partial(
        pl.kernel,
        mesh=mesh,
        compiler_params=pltpu.CompilerParams(
            use_tc_tiling_on_sc=False, needs_layout_passes=False
        ),
        out_type=jax.ShapeDtypeStruct((H, E8, B // _LB, 8, _LB), jnp.float32),
        scratch_types=[
            pltpu.VMEM((H, b_per_w), jnp.int32),
            pltpu.VMEM((2, _LB, D), jnp.float32),
            pltpu.VMEM((2, E8, 8, _LB), jnp.float32),
            pltpu.SemaphoreType.DMA,
            pltpu.SemaphoreType.DMA,
            pltpu.SemaphoreType.DMA,
            pltpu.SemaphoreType.DMA,
        ],
    )
    def gather(xt_hbm, table_hbm, out_hbm, idx_v, rows_v, t_v, g0, g1, s0, s1):
        gsem = (g0, g1)
        ssem = (s0, s1)
        wid = lax.axis_index("s") * NC + lax.axis_index("c")
        bbase = wid * blk_per_w
        pltpu.sync_copy(
            xt_hbm.at[:, pl.ds(pl.multiple_of(wid * b_per_w, b_per_w), b_per_w)],
            idx_v,
        )

        lane = lax.iota(jnp.int32, 16)
        row_idx = [lane + (g * 16) for g in range(_LB // 16)]

        def tile_coords(t):
            h = lax.shift_right_logical(t, 2)
            blk = lax.bitwise_and(t, blk_per_w - 1)
            return h, blk

        def fire_gather(t, b):
            h, blk = tile_coords(t)
            pltpu.async_copy(
                table_hbm.at[idx_v.at[h, pl.ds(pl.multiple_of(blk * _LB, _LB), _LB)]],
                rows_v.at[b],
                gsem[b],
            )

        def wait_gather(b):
            pltpu.make_async_copy(
                table_hbm.at[pl.ds(0, _LB)], rows_v.at[b], gsem[b]
            ).wait()

        def transpose(b):
            @plsc.parallel_loop(0, D, 1, unroll=8)
            def _(e):
                e8 = lax.shift_right_logical(e, 3)
                es = lax.bitwise_and(e, 7)
                ce = jnp.full((16,), e, jnp.int32)
                for g in range(_LB // 16):
                    v = plsc.load_gather(rows_v.at[b], [row_idx[g], ce])
                    t_v[b, e8, es, pl.ds(g * 16, 16)] = v

        def fire_store(t, b):
            h, blk = tile_coords(t)
            pltpu.async_copy(t_v.at[b], out_hbm.at[h, :, bbase + blk], ssem[b])

        def wait_store(b):
            pltpu.make_async_copy(
                t_v.at[b], out_hbm.at[0, :, 0], ssem[b]
            ).wait()

        fire_gather(0, 0)

        def pair(i, carry):
            t = pl.multiple_of(i * 2, 2)
            wait_gather(0)
            fire_gather(t + 1, 1)

            @pl.when(i > 0)
            def _():
                wait_store(0)

            transpose(0)
            fire_store(t, 0)

            wait_gather(1)

            @pl.when(i < n_pairs - 1)
            def _():
                fire_gather(t + 2, 0)

            @pl.when(i > 0)
            def _():
                wait_store(1)

            transpose(1)
            fire_store(t + 1, 1)
            return carry

        lax.fori_loop(0, n_pairs, pair, 0)
        wait_store(0)
        wait_store(1)

    return gather


def kernel(x, table):
    B, H = x.shape
    V, D = table.shape
    xt = x.T.astype(jnp.int32)
    info = plsc.get_sparse_core_info()
    gather = _make_gather(B, H, V, D, info.num_cores, info.num_subcores)
    out5 = gather(xt, table)
    return out5.transpose(2, 4, 0, 1, 3).reshape(B, H, D)


# speedup vs baseline: 1.4118x; 1.0044x over previous
import functools

import jax
import jax.numpy as jnp
from jax import lax
from jax.experimental import pallas as pl
from jax.experimental.pallas import tpu as pltpu
from jax.experimental.pallas import tpu_sc as plsc

_LB = 128


def _make_gather(B, H, V, D, NC, NS):
    NW = NC * NS
    blk_per_w = B // _LB // NW
    b_per_w = blk_per_w * _LB
    n_tiles = H * blk_per_w
    n_pairs = n_tiles // 2
    E8 = D // 8

    mesh = plsc.VectorSubcoreMesh(core_axis_name="c", subcore_axis_name="s")

    @functools.partial(
        pl.kernel,
        mesh=mesh,
        compiler_params=pltpu.CompilerParams(
            use_tc_tiling_on_sc=False, needs_layout_passes=False
        ),
        out_type=jax.ShapeDtypeStruct((H, E8, B // _LB, 8, _LB), jnp.float32),
        scratch_types=[
            pltpu.VMEM((H, b_per_w), jnp.int32),
            pltpu.VMEM((4, _LB, D), jnp.float32),
            pltpu.VMEM((2, E8, 8, _LB), jnp.float32),
            pltpu.SemaphoreType.DMA,
            pltpu.SemaphoreType.DMA,
            pltpu.SemaphoreType.DMA,
            pltpu.SemaphoreType.DMA,
            pltpu.SemaphoreType.DMA,
            pltpu.SemaphoreType.DMA,
        ],
    )
    def gather(xt_hbm, table_hbm, out_hbm, idx_v, rows_v, t_v,
               g0, g1, g2, g3, s0, s1):
        gsem = (g0, g1, g2, g3)
        ssem = (s0, s1)
        wid = lax.axis_index("s") * NC + lax.axis_index("c")
        bbase = wid * blk_per_w
        pltpu.sync_copy(
            xt_hbm.at[:, pl.ds(pl.multiple_of(wid * b_per_w, b_per_w), b_per_w)],
            idx_v,
        )

        lane = lax.iota(jnp.int32, 16)
        row_idx = [lane + (g * 16) for g in range(_LB // 16)]

        def tile_coords(t):
            h = lax.shift_right_logical(t, 2)
            blk = lax.bitwise_and(t, blk_per_w - 1)
            return h, blk

        def fire_gather(t, b):
            h, blk = tile_coords(t)
            pltpu.async_copy(
                table_hbm.at[idx_v.at[h, pl.ds(pl.multiple_of(blk * _LB, _LB), _LB)]],
                rows_v.at[b],
                gsem[b],
            )

        def wait_gather(b):
            pltpu.make_async_copy(
                table_hbm.at[pl.ds(0, _LB)], rows_v.at[b], gsem[b]
            ).wait()

        def transpose(b, tb):
            @plsc.parallel_loop(0, D, 1, unroll=16)
            def _(e):
                e8 = lax.shift_right_logical(e, 3)
                es = lax.bitwise_and(e, 7)
                ce = jnp.full((16,), e, jnp.int32)
                for g in range(_LB // 16):
                    v = plsc.load_gather(rows_v.at[b], [row_idx[g], ce])
                    t_v[tb, e8, es, pl.ds(g * 16, 16)] = v

        def fire_store(t, b):
            h, blk = tile_coords(t)
            pltpu.async_copy(t_v.at[b], out_hbm.at[h, :, bbase + blk], ssem[b])

        def wait_store(b):
            pltpu.make_async_copy(
                t_v.at[b], out_hbm.at[0, :, 0], ssem[b]
            ).wait()

        fire_gather(0, 0)
        fire_gather(1, 1)
        fire_gather(2, 2)

        n_quads = n_tiles // 4

        def quad(i, carry):
            t = pl.multiple_of(i * 4, 4)
            for b in range(4):
                tt = t + b
                tb = b % 2
                wait_gather(b)

                if b == 0:
                    fire_gather(tt + 3, 3)
                else:
                    @pl.when(i <= n_quads - 2)
                    def _():
                        fire_gather(tt + 3, (b + 3) % 4)

                if b < 2:
                    @pl.when(i > 0)
                    def _():
                        wait_store(tb)
                else:
                    wait_store(tb)

                transpose(b, tb)
                fire_store(tt, tb)
            return carry

        lax.fori_loop(0, n_quads, quad, 0)
        wait_store(0)
        wait_store(1)

    return gather


def kernel(x, table):
    B, H = x.shape
    V, D = table.shape
    xt = x.T.astype(jnp.int32)
    info = plsc.get_sparse_core_info()
    gather = _make_gather(B, H, V, D, info.num_cores, info.num_subcores)
    out5 = gather(xt, table)
    return out5.transpose(2, 4, 0, 1, 3).reshape(B, H, D)
